# baseline (speedup 1.0000x reference)
import jax
import jax.numpy as jnp
from jax import lax
from jax.experimental import pallas as pl
from jax.experimental.pallas import tpu as pltpu

_BETA = 64
_ALPHA = 2
_G = 8


def _lga_block_kernel(kxyz_ref, lc_ref, kx_ref, sph_ref, pct_ref, dir_ref, feat_ref):
    kxyz3 = kxyz_ref[0]
    lc = lc_ref[0]
    sph = sph_ref[...]
    T, K, _ = kxyz3.shape
    C = kx_ref.shape[3]
    R = T * K

    rel = (kxyz3 - lc[:, None, :]).reshape(R, 3)
    r0 = rel[:, 0:1]
    r1 = rel[:, 1:2]
    r2 = rel[:, 2:3]
    dist = jnp.sqrt(r0 * r0 + r1 * r1 + r2 * r2)
    den = dist + 1e-08
    nrm = (rel / den).astype(jnp.bfloat16)

    u = jax.lax.dot_general(
        nrm, sph.astype(jnp.bfloat16),
        (((1,), (0,)), ((), ())),
        preferred_element_type=jnp.float32,
    )

    umax = jnp.max(u, axis=-1, keepdims=True)
    lane_a = lax.broadcasted_iota(jnp.int32, (R, _BETA), 1)
    idx = jnp.min(jnp.where(u >= umax, lane_a, _BETA), axis=-1, keepdims=True)
    oh = (lane_a == idx).astype(jnp.float32)

    oh3 = oh.reshape(T, K, _BETA)
    counts = jnp.sum(oh3, axis=1)
    bin_iota = lax.broadcasted_iota(jnp.int32, (T, _BETA), 1)
    counts = counts - (bin_iota == 0).astype(jnp.float32)
    thr = jnp.where(counts > _ALPHA, counts, 0.0)
    pct_ref[0] = thr / (jnp.sum(thr, axis=-1, keepdims=True) + 1e-08)

    cden = counts + 1e-08
    relb = rel.astype(jnp.bfloat16).astype(jnp.float32)
    for s in range(3):
        rs3 = relb[:, s:s + 1].reshape(T, K, 1)
        dir_ref[:, s, :] = jnp.sum(oh3 * rs3, axis=1) / cden

    kxb = kx_ref[0].reshape(R, C).astype(jnp.bfloat16)

    GK = _G * K
    GA = _G * _BETA
    col_iota = lax.broadcasted_iota(jnp.int32, (GK, GA), 1)
    row_off = (lax.broadcasted_iota(jnp.int32, (GK, 1), 0) // K) * _BETA

    ngroups = T // _G
    for g in range(ngroups):
        idx_g = idx[g * GK:(g + 1) * GK, :]
        ohbd = jnp.where(
            col_iota == row_off + idx_g, 1.0, 0.0
        ).astype(jnp.bfloat16)
        feats = jax.lax.dot_general(
            ohbd, kxb[g * GK:(g + 1) * GK, :],
            (((0,), (0,)), ((), ())),
            preferred_element_type=jnp.float32,
        )
        fden = jnp.sum(feats, axis=-1, keepdims=True) + 1e-09
        feat_ref[0, g * _G:(g + 1) * _G, :, :] = (feats / fden).reshape(_G, _BETA, C)


def kernel(lc_xyz, lc_x, knn_xyz, knn_x, sphere_points):
    B, N, K, C = knn_x.shape
    M = B * N
    T = 128
    nb = N // T

    sph = sphere_points.T

    pct, dirs, feat = pl.pallas_call(
        _lga_block_kernel,
        grid=(M // T,),
        in_specs=[
            pl.BlockSpec((1, T, K, 3), lambda i: (i // nb, i % nb, 0, 0)),
            pl.BlockSpec((1, T, 3), lambda i: (i // nb, i % nb, 0)),
            pl.BlockSpec((1, T, K, C), lambda i: (i // nb, i % nb, 0, 0)),
            pl.BlockSpec((3, _BETA), lambda i: (0, 0)),
        ],
        out_specs=[
            pl.BlockSpec((1, T, _BETA), lambda i: (i // nb, i % nb, 0)),
            pl.BlockSpec((T, 3, _BETA), lambda i: (i, 0, 0)),
            pl.BlockSpec((1, T, _BETA, C), lambda i: (i // nb, i % nb, 0, 0)),
        ],
        out_shape=[
            jax.ShapeDtypeStruct((B, N, _BETA), jnp.float32),
            jax.ShapeDtypeStruct((M, 3, _BETA), jnp.float32),
            jax.ShapeDtypeStruct((B, N, _BETA, C), jnp.float32),
        ],
        compiler_params=pltpu.CompilerParams(
            dimension_semantics=("arbitrary",),
        ),
    )(knn_xyz, lc_xyz, knn_x, sph)

    avg_direction = jnp.transpose(dirs.reshape(B, N, 3, _BETA), (0, 1, 3, 2))
    k_influence = jnp.ones((B, N), jnp.float32)
    return (knn_x, pct, avg_direction, feat, k_influence)


# speedup vs baseline: 1.1235x; 1.1235x over previous
import jax
import jax.numpy as jnp
from jax import lax
from jax.experimental import pallas as pl
from jax.experimental.pallas import tpu as pltpu

_BETA = 64
_ALPHA = 2
_G = 8


def _lga_block_kernel(kxyz_ref, lc_ref, kx_ref, sph_ref, pct_ref, dir_ref, feat_ref):
    kxyz3 = kxyz_ref[0]
    lc = lc_ref[0]
    sph = sph_ref[...]
    T, K, _ = kxyz3.shape
    C = kx_ref.shape[3]
    R = T * K

    rel = (kxyz3 - lc[:, None, :]).reshape(R, 3)
    r0 = rel[:, 0:1]
    r1 = rel[:, 1:2]
    r2 = rel[:, 2:3]
    dist = jnp.sqrt(r0 * r0 + r1 * r1 + r2 * r2)
    den = dist + 1e-08
    nrm = (rel / den).astype(jnp.bfloat16)

    u = jax.lax.dot_general(
        nrm, sph.astype(jnp.bfloat16),
        (((1,), (0,)), ((), ())),
        preferred_element_type=jnp.float32,
    )

    umax = jnp.max(u, axis=-1, keepdims=True)
    lane_a = lax.broadcasted_iota(jnp.int32, (R, _BETA), 1)
    idx = jnp.min(jnp.where(u >= umax, lane_a, _BETA), axis=-1, keepdims=True)
    oh = (lane_a == idx).astype(jnp.float32)

    oh3 = oh.reshape(T, K, _BETA)
    counts = jnp.sum(oh3, axis=1)
    bin_iota = lax.broadcasted_iota(jnp.int32, (T, _BETA), 1)
    counts = counts - (bin_iota == 0).astype(jnp.float32)
    thr = jnp.where(counts > _ALPHA, counts, 0.0)
    pct_ref[0] = thr / (jnp.sum(thr, axis=-1, keepdims=True) + 1e-08)

    cden = counts + 1e-08
    relb = rel.astype(jnp.bfloat16).astype(jnp.float32)
    for s in range(3):
        rs3 = relb[:, s:s + 1].reshape(T, K, 1)
        dir_ref[:, s, :] = jnp.sum(oh3 * rs3, axis=1) / cden

    kxb = kx_ref[0].reshape(R, C).astype(jnp.bfloat16)

    GK = _G * K
    GA = _G * _BETA
    col_iota = lax.broadcasted_iota(jnp.int32, (GK, GA), 1)
    row_off = (lax.broadcasted_iota(jnp.int32, (GK, 1), 0) // K) * _BETA

    ngroups = T // _G
    for g in range(ngroups):
        idx_g = idx[g * GK:(g + 1) * GK, :]
        ohbd = jnp.where(
            col_iota == row_off + idx_g, 1.0, 0.0
        ).astype(jnp.bfloat16)
        feats = jax.lax.dot_general(
            ohbd, kxb[g * GK:(g + 1) * GK, :],
            (((0,), (0,)), ((), ())),
            preferred_element_type=jnp.float32,
        )
        fden = jnp.sum(feats, axis=-1, keepdims=True) + 1e-09
        feat_ref[g * _G:(g + 1) * _G, :, :] = (feats / fden).reshape(_G, _BETA, C)


def kernel(lc_xyz, lc_x, knn_xyz, knn_x, sphere_points):
    B, N, K, C = knn_x.shape
    M = B * N
    T = 128
    nb = N // T

    sph = sphere_points.T

    pct, dirs, feat = pl.pallas_call(
        _lga_block_kernel,
        grid=(M // T,),
        in_specs=[
            pl.BlockSpec((1, T, K, 3), lambda i: (i // nb, i % nb, 0, 0)),
            pl.BlockSpec((1, T, 3), lambda i: (i // nb, i % nb, 0)),
            pl.BlockSpec((1, T, K, C), lambda i: (i // nb, i % nb, 0, 0)),
            pl.BlockSpec((3, _BETA), lambda i: (0, 0)),
        ],
        out_specs=[
            pl.BlockSpec((1, T, _BETA), lambda i: (i // nb, i % nb, 0)),
            pl.BlockSpec((T, 3, _BETA), lambda i: (i, 0, 0)),
            pl.BlockSpec((T, _BETA, C), lambda i: (i, 0, 0)),
        ],
        out_shape=[
            jax.ShapeDtypeStruct((B, N, _BETA), jnp.float32),
            jax.ShapeDtypeStruct((M, 3, _BETA), jnp.float32),
            jax.ShapeDtypeStruct((M, _BETA, C), jnp.float32),
        ],
        compiler_params=pltpu.CompilerParams(
            dimension_semantics=("arbitrary",),
        ),
    )(knn_xyz, lc_xyz, knn_x, sph)

    avg_direction = jnp.transpose(dirs.reshape(B, N, 3, _BETA), (0, 1, 3, 2))
    avg_features = feat.reshape(B, N, _BETA, C)
    k_influence = jnp.ones((B, N), jnp.float32)
    return (knn_x, pct, avg_direction, avg_features, k_influence)
